# Initial kernel scaffold; baseline (speedup 1.0000x reference)
import jax
import jax.numpy as jnp
from jax.experimental import pallas as pl


def kernel(x, _input, table, W, b):
    raise NotImplementedError("write your pallas kernel here")


# speedup vs baseline: 36.7884x; 36.7884x over previous
import functools

import jax
import jax.numpy as jnp
from jax import lax
from jax.experimental import pallas as pl
from jax.experimental.pallas import tpu as pltpu
from jax.experimental.pallas import tpu_sc as plsc

BS = 4096
N_DIST = 20
N_FEAT = 26
DIM = 32
NH = 128

ROWS = BS * N_DIST
TOTAL_IDX = ROWS * N_FEAT

NW = 32
IDX_PER_W = TOTAL_IDX // NW
GL = 128
K = 8
NCH = IDX_PER_W // (K * GL)


def _sc_gather(idx4, table):
    mesh = plsc.VectorSubcoreMesh(core_axis_name="c", subcore_axis_name="s")

    @functools.partial(
        pl.kernel,
        out_type=jax.ShapeDtypeStruct((NW, NCH, K, GL, DIM), jnp.float32),
        mesh=mesh,
        compiler_params=pltpu.CompilerParams(use_tc_tiling_on_sc=False),
        scratch_types=[
            pltpu.VMEM((K, GL), jnp.int32),
            pltpu.VMEM((K, GL, DIM), jnp.float32),
            pltpu.SemaphoreType.DMA,
        ],
    )
    def body(idx_hbm, table_hbm, out_hbm, idx_v, rows_v, sem):
        wid = lax.axis_index("s") * 2 + lax.axis_index("c")

        @pl.loop(0, NCH)
        def _chunk(ch):
            pltpu.sync_copy(idx_hbm.at[wid, ch], idx_v)
            descs = [
                pltpu.async_copy(table_hbm.at[idx_v.at[j]], rows_v.at[j], sem)
                for j in range(K)
            ]
            for d in descs:
                d.wait()
            pltpu.sync_copy(rows_v, out_hbm.at[wid, ch])

    return body(idx4, table)


def _tc_score(g, x, idx, W, b):
    R = 1280
    BB = R // N_DIST

    def body(g_ref, x_ref, idx_ref, w_ref, b_ref, o_ref):
        gb = g_ref[...].astype(jnp.bfloat16)
        wb = w_ref[...].astype(jnp.bfloat16)
        z = jnp.dot(gb, wb, preferred_element_type=jnp.float32) + b_ref[...]
        h = jnp.tanh(z)
        hr = h.reshape(BB, N_DIST, NH)
        xb = x_ref[...]
        dots = jnp.sum(hr * xb[:, None, :], axis=-1)
        mask = jnp.all(idx_ref[...] == 0, axis=-1)
        o_ref[...] = jnp.where(mask, -jnp.inf, dots)

    return pl.pallas_call(
        body,
        grid=(ROWS // R,),
        in_specs=[
            pl.BlockSpec((R, N_FEAT * DIM), lambda i: (i, 0)),
            pl.BlockSpec((BB, NH), lambda i: (i, 0)),
            pl.BlockSpec((BB, N_DIST, N_FEAT), lambda i: (i, 0, 0)),
            pl.BlockSpec((N_FEAT * DIM, NH), lambda i: (0, 0)),
            pl.BlockSpec((1, NH), lambda i: (0, 0)),
        ],
        out_specs=pl.BlockSpec((BB, N_DIST), lambda i: (i, 0)),
        out_shape=jax.ShapeDtypeStruct((BS, N_DIST), jnp.float32),
    )(g, x, idx, W, b)


def kernel(x, _input, table, W, b):
    idx4 = _input.reshape(NW, NCH, K, GL)
    g = _sc_gather(idx4, table).reshape(ROWS, N_FEAT * DIM)
    return _tc_score(g, x, _input, W, b.reshape(1, NH))
